# baseline (speedup 1.0000x reference)
import jax
import jax.numpy as jnp
from jax import lax
from jax.experimental import pallas as pl
from jax.experimental.pallas import tpu as pltpu

_C = 32
_NBUF = 4


def _add_body(x_hbm, p_ref, o_hbm, in_bufs, out_bufs, in_sems, out_sems):
    n_chunks = x_hbm.shape[0] // _C

    def in_copy(i, slot):
        return pltpu.make_async_copy(
            x_hbm.at[pl.ds(i * _C, _C)], in_bufs.at[slot], in_sems.at[slot]
        )

    def out_copy(i, slot):
        return pltpu.make_async_copy(
            out_bufs.at[slot], o_hbm.at[pl.ds(i * _C, _C)], out_sems.at[slot]
        )

    for s in range(_NBUF):
        in_copy(s, s).start()

    pos = p_ref[...]

    def step(i, carry):
        slot = lax.rem(i, _NBUF)
        in_copy(i, slot).wait()

        @pl.when(i >= _NBUF)
        def _():
            out_copy(i - _NBUF, slot).wait()

        out_bufs[slot] = in_bufs[slot] + pos
        out_copy(i, slot).start()

        @pl.when(i + _NBUF < n_chunks)
        def _():
            in_copy(i + _NBUF, slot).start()

        return carry

    lax.fori_loop(0, n_chunks, step, 0)

    for s in range(_NBUF):
        i = n_chunks - _NBUF + s
        out_copy(i, lax.rem(i, _NBUF)).wait()


def kernel(x, pos_emb):
    B, S, D = x.shape
    out = pl.pallas_call(
        _add_body,
        grid=(),
        in_specs=[
            pl.BlockSpec(memory_space=pl.ANY),
            pl.BlockSpec(memory_space=pltpu.VMEM),
        ],
        out_specs=pl.BlockSpec(memory_space=pl.ANY),
        out_shape=jax.ShapeDtypeStruct((B, S, D), x.dtype),
        scratch_shapes=[
            pltpu.VMEM((_NBUF, _C, S, D), x.dtype),
            pltpu.VMEM((_NBUF, _C, S, D), x.dtype),
            pltpu.SemaphoreType.DMA((_NBUF,)),
            pltpu.SemaphoreType.DMA((_NBUF,)),
        ],
    )(x, pos_emb[None])
    return out


# speedup vs baseline: 6.0844x; 6.0844x over previous
import jax
import jax.numpy as jnp
from jax.experimental import pallas as pl


def _add_body(x_ref, p_ref, o_ref):
    o_ref[...] = x_ref[...] + p_ref[...]


def kernel(x, pos_emb):
    B, S, D = x.shape
    xt = jnp.transpose(x, (1, 2, 0))
    pt = pos_emb.reshape(S, D, 1)
    SB = 8
    out_t = pl.pallas_call(
        _add_body,
        grid=(S // SB,),
        in_specs=[
            pl.BlockSpec((SB, D, B), lambda i: (i, 0, 0)),
            pl.BlockSpec((SB, D, 1), lambda i: (i, 0, 0)),
        ],
        out_specs=pl.BlockSpec((SB, D, B), lambda i: (i, 0, 0)),
        out_shape=jax.ShapeDtypeStruct((S, D, B), x.dtype),
    )(xt, pt)
    return jnp.transpose(out_t, (2, 0, 1))


# speedup vs baseline: 6.3463x; 1.0430x over previous
import jax
import jax.numpy as jnp
from jax.experimental import pallas as pl


def _make_body(SB, D, B):
    def _add_body(x_ref, p_ref, o_ref):
        p = jax.lax.broadcast_in_dim(p_ref[...], (SB, D, B), (0, 1))
        o_ref[...] = x_ref[...] + p
    return _add_body


def kernel(x, pos_emb):
    B, S, D = x.shape
    xt = jnp.transpose(x, (1, 2, 0))
    SB = 8
    out_t = pl.pallas_call(
        _make_body(SB, D, B),
        grid=(S // SB,),
        in_specs=[
            pl.BlockSpec((SB, D, B), lambda i: (i, 0, 0)),
            pl.BlockSpec((SB, D), lambda i: (i, 0)),
        ],
        out_specs=pl.BlockSpec((SB, D, B), lambda i: (i, 0, 0)),
        out_shape=jax.ShapeDtypeStruct((S, D, B), x.dtype),
    )(xt, pos_emb)
    return jnp.transpose(out_t, (2, 0, 1))
